# baseline (speedup 1.0000x reference)
import jax
import jax.numpy as jnp
from jax.experimental import pallas as pl

_N = 48
_L = 64
_STEPS = _N - 1
_SLABS = 2


def _leaky(x):
    return jnp.maximum(x, 0.01 * x)


def _db(a, b):
    return jax.lax.dot_general(
        a, b, (((1,), (0,)), ((), ())), preferred_element_type=jnp.float32)


def _sigmoid(x):
    return 0.5 * jnp.tanh(0.5 * x) + 0.5


def _prims_kernel(x0m_ref, x1m_ref, x0s_ref, x1s_ref,
                  enc_w_ref, enc_b_ref, m1w_ref, m2w_ref, uw_ref,
                  mw_ref, mb_ref, p1w_ref, pb1_ref, p2w_ref, pb2_ref,
                  out_ref):
    f32 = jnp.float32
    bf16 = jnp.bfloat16
    E = _N * _N

    d0 = x0m_ref[:, :, :] - x0s_ref[:, :, :]
    d1 = x1m_ref[:, :, :] - x1s_ref[:, :, :]
    ew3 = jnp.sqrt(d0 * d0 + d1 * d1 + 1e-12)
    ewcol = ew3.astype(bf16).reshape(E, 1)

    enc_w = enc_w_ref[:, :].astype(bf16)
    enc_b = enc_b_ref[:, :]
    m1w = m1w_ref[:, :].astype(bf16)
    m2w = m2w_ref[:, :].astype(bf16)
    uw = uw_ref[:, :].astype(bf16)
    mw = mw_ref[:, :].astype(bf16)
    mb = mb_ref[:, :]

    iota = jax.lax.broadcasted_iota(jnp.int32, (_N, 1), 0)

    def step(_, carry):
        h, pt, _enc = carry
        enc_in = jnp.concatenate([pt.astype(bf16), h.astype(bf16)], axis=1)
        encoded = jnp.maximum(_db(enc_in, enc_w) + enc_b, 0.0)
        encb = encoded.astype(bf16)
        ns = _N // _SLABS
        parts = []
        for s in range(_SLABS):
            e_dst = jnp.broadcast_to(
                encb[None, :, :], (ns, _N, _L)).reshape(ns * _N, _L)
            e_src = jnp.broadcast_to(
                encb[s * ns:(s + 1) * ns, None, :],
                (ns, _N, _L)).reshape(ns * _N, _L)
            op = jnp.concatenate(
                [e_dst, e_src, ewcol[s * ns * _N:(s + 1) * ns * _N]], axis=1)
            z = _db(_leaky(_db(op, m1w)).astype(bf16), m2w)
            parts.append(jnp.max(z.reshape(ns, _N, _L), axis=0))
        mx_z = parts[0]
        for p in parts[1:]:
            mx_z = jnp.maximum(mx_z, p)
        aggr = _leaky(mx_z)
        h_new = jnp.clip(
            _leaky(_db(jnp.concatenate([encb, aggr.astype(bf16)], axis=1),
                       uw)), -1e9, 1e9)
        logits = _sigmoid(
            _db(jnp.concatenate([encb, h_new.astype(bf16)], axis=1), mw) + mb)
        mx = jnp.max(logits)
        idx = jnp.min(jnp.where(logits == mx, iota, _N))
        pt_new = jnp.where(iota == idx, 1.0, pt)
        return (h_new, pt_new, encoded)

    init = (jnp.zeros((_N, _L), f32),
            jnp.zeros((_N, 1), f32),
            jnp.zeros((_N, _L), f32))
    h, _pt, enc = jax.lax.fori_loop(0, _STEPS, step, init)

    eh = jnp.concatenate([enc.astype(bf16), h.astype(bf16)], axis=1)
    s_part = _db(eh, p1w_ref[0:2 * _L, :].astype(bf16))
    d_part = _db(eh, p1w_ref[2 * _L:, :].astype(bf16))
    pe = jnp.maximum(
        s_part[:, None, :] + d_part[None, :, :] + pb1_ref[:, :][None, :, :],
        0.0)
    out_ref[:, :] = _db(pe.reshape(E, _L).astype(bf16),
                        p2w_ref[:, :].astype(bf16)) + pb2_ref[:, :]


def kernel(X, enc_W, enc_b, M1_W, M2_W, U_W, mst_W, mst_b,
           pred_W1, pred_b1, pred_W2, pred_b2):
    args = (
        X[:, 0].reshape(_N, 1, 1), X[:, 1].reshape(_N, 1, 1),
        X[:, 0].reshape(1, _N, 1), X[:, 1].reshape(1, _N, 1),
        enc_W, enc_b.reshape(1, _L),
        M1_W, M2_W, U_W,
        mst_W, mst_b.reshape(1, 1),
        pred_W1, pred_b1.reshape(1, _L),
        pred_W2, pred_b2.reshape(1, 1),
    )
    out = pl.pallas_call(
        _prims_kernel,
        out_shape=jax.ShapeDtypeStruct((_N * _N, 1), jnp.float32),
    )(*args)
    return out.reshape(_N, _N)


# speedup vs baseline: 1.1039x; 1.1039x over previous
import jax
import jax.numpy as jnp
from jax.experimental import pallas as pl
from jax.experimental.pallas import tpu as pltpu

_N = 48
_L = 64
_STEPS = _N - 1


def _leaky(x):
    return jnp.maximum(x, 0.01 * x)


def _db(a, b):
    return jax.lax.dot_general(
        a, b, (((1,), (0,)), ((), ())), preferred_element_type=jnp.float32)


def _sigmoid(x):
    return 0.5 * jnp.tanh(0.5 * x) + 0.5


def _prims_kernel(x0m_ref, x1m_ref, x0s_ref, x1s_ref,
                  enc_w_ref, enc_b_ref, m1w_ref, m2w_ref, uw_ref,
                  mw_ref, mb_ref, p1w_ref, pb1_ref, p2w_ref, pb2_ref,
                  out_ref, op_ref):
    f32 = jnp.float32
    bf16 = jnp.bfloat16
    E = _N * _N

    d0 = x0m_ref[:, :, :] - x0s_ref[:, :, :]
    d1 = x1m_ref[:, :, :] - x1s_ref[:, :, :]
    ew3 = jnp.sqrt(d0 * d0 + d1 * d1 + 1e-12)
    op_ref[:, 2 * _L:] = ew3.astype(bf16).reshape(E, 1)

    enc_w = enc_w_ref[:, :].astype(bf16)
    enc_b = enc_b_ref[:, :]
    m1w = m1w_ref[:, :].astype(bf16)
    m2w = m2w_ref[:, :].astype(bf16)
    uw = uw_ref[:, :].astype(bf16)
    mw = mw_ref[:, :].astype(bf16)
    mb = mb_ref[:, :]

    iota = jax.lax.broadcasted_iota(jnp.int32, (_N, 1), 0)

    def step(_, carry):
        h, pt, _enc = carry
        enc_in = jnp.concatenate([pt.astype(bf16), h.astype(bf16)], axis=1)
        encoded = jnp.maximum(_db(enc_in, enc_w) + enc_b, 0.0)
        encb = encoded.astype(bf16)
        op_ref[:, 0:_L] = jnp.broadcast_to(
            encb[None, :, :], (_N, _N, _L)).reshape(E, _L)
        op_ref[:, _L:2 * _L] = jnp.broadcast_to(
            encb[:, None, :], (_N, _N, _L)).reshape(E, _L)
        m1 = _leaky(_db(op_ref[:, :], m1w))
        z = _db(m1.astype(bf16), m2w)
        aggr = _leaky(jnp.max(z.reshape(_N, _N, _L), axis=0))
        h_new = jnp.clip(
            _leaky(_db(jnp.concatenate([encb, aggr.astype(bf16)], axis=1),
                       uw)), -1e9, 1e9)
        logits = _sigmoid(
            _db(jnp.concatenate([encb, h_new.astype(bf16)], axis=1), mw) + mb)
        mx = jnp.max(logits)
        idx = jnp.min(jnp.where(logits == mx, iota, _N))
        pt_new = jnp.where(iota == idx, 1.0, pt)
        return (h_new, pt_new, encoded)

    init = (jnp.zeros((_N, _L), f32),
            jnp.zeros((_N, 1), f32),
            jnp.zeros((_N, _L), f32))
    h, _pt, enc = jax.lax.fori_loop(0, _STEPS, step, init)

    eh = jnp.concatenate([enc.astype(bf16), h.astype(bf16)], axis=1)
    s_part = _db(eh, p1w_ref[0:2 * _L, :].astype(bf16))
    d_part = _db(eh, p1w_ref[2 * _L:, :].astype(bf16))
    pe = jnp.maximum(
        s_part[:, None, :] + d_part[None, :, :] + pb1_ref[:, :][None, :, :],
        0.0)
    out_ref[:, :] = _db(pe.reshape(E, _L).astype(bf16),
                        p2w_ref[:, :].astype(bf16)) + pb2_ref[:, :]


def kernel(X, enc_W, enc_b, M1_W, M2_W, U_W, mst_W, mst_b,
           pred_W1, pred_b1, pred_W2, pred_b2):
    args = (
        X[:, 0].reshape(_N, 1, 1), X[:, 1].reshape(_N, 1, 1),
        X[:, 0].reshape(1, _N, 1), X[:, 1].reshape(1, _N, 1),
        enc_W, enc_b.reshape(1, _L),
        M1_W, M2_W, U_W,
        mst_W, mst_b.reshape(1, 1),
        pred_W1, pred_b1.reshape(1, _L),
        pred_W2, pred_b2.reshape(1, 1),
    )
    out = pl.pallas_call(
        _prims_kernel,
        out_shape=jax.ShapeDtypeStruct((_N * _N, 1), jnp.float32),
        scratch_shapes=[pltpu.VMEM((_N * _N, 2 * _L + 1), jnp.bfloat16)],
    )(*args)
    return out.reshape(_N, _N)


# speedup vs baseline: 1.1293x; 1.0231x over previous
import jax
import jax.numpy as jnp
from jax.experimental import pallas as pl
from jax.experimental.pallas import tpu as pltpu

_N = 48
_L = 64
_STEPS = _N - 1


def _leaky(x):
    return jnp.maximum(x, 0.01 * x)


def _db(a, b):
    return jax.lax.dot_general(
        a, b, (((1,), (0,)), ((), ())), preferred_element_type=jnp.float32)


def _sigmoid(x):
    return 0.5 * jnp.tanh(0.5 * x) + 0.5


def _prims_kernel(x0m_ref, x1m_ref, x0s_ref, x1s_ref,
                  enc_w_ref, enc_b_ref, m1w_ref, m2w_ref, uw_ref,
                  mw_ref, mb_ref, p1w_ref, pb1_ref, p2w_ref, pb2_ref,
                  out_ref, op_ref):
    f32 = jnp.float32
    bf16 = jnp.bfloat16
    E = _N * _N

    d0 = x0m_ref[:, :, :] - x0s_ref[:, :, :]
    d1 = x1m_ref[:, :, :] - x1s_ref[:, :, :]
    ew3 = jnp.sqrt(d0 * d0 + d1 * d1 + 1e-12)
    op_ref[:, 2 * _L:] = ew3.astype(bf16).reshape(E, 1)

    enc_w = enc_w_ref[:, :].astype(bf16)
    enc_b = enc_b_ref[:, :]
    m1w = m1w_ref[:, :].astype(bf16)
    m2w = m2w_ref[:, :].astype(bf16)
    uw = uw_ref[:, :].astype(bf16)
    mw = mw_ref[:, :].astype(bf16)
    mb = mb_ref[:, :]

    iota = jax.lax.broadcasted_iota(jnp.int32, (_N, 1), 0)
    zcol = jnp.zeros((_N, 1), bf16)
    ocol = jnp.ones((_N, 1), bf16)

    h0 = jnp.zeros((_N, _L), f32)
    pt0 = jnp.zeros((_N, 1), f32)
    enc0 = jnp.maximum(
        _db(jnp.concatenate([zcol, h0.astype(bf16)], axis=1), enc_w) + enc_b,
        0.0)

    def step(_, carry):
        h, pt, encoded, _enc_prev = carry
        encb = encoded.astype(bf16)
        op_ref[:, 0:_L] = jnp.broadcast_to(
            encb[None, :, :], (_N, _N, _L)).reshape(E, _L)
        op_ref[:, _L:2 * _L] = jnp.broadcast_to(
            encb[:, None, :], (_N, _N, _L)).reshape(E, _L)
        m1 = _leaky(_db(op_ref[:, :], m1w))
        z = _db(m1.astype(bf16), m2w)
        aggr = _leaky(jnp.max(z.reshape(_N, _N, _L), axis=0))
        h_new = jnp.clip(
            _leaky(_db(jnp.concatenate([encb, aggr.astype(bf16)], axis=1),
                       uw)), -1e9, 1e9)
        hb = h_new.astype(bf16)
        logits = _sigmoid(
            _db(jnp.concatenate([encb, hb], axis=1), mw) + mb)
        e_if0 = jnp.maximum(
            _db(jnp.concatenate([zcol, hb], axis=1), enc_w) + enc_b, 0.0)
        e_if1 = jnp.maximum(
            _db(jnp.concatenate([ocol, hb], axis=1), enc_w) + enc_b, 0.0)
        mx = jnp.max(logits)
        idx = jnp.min(jnp.where(logits == mx, iota, _N))
        pt_new = jnp.where(iota == idx, 1.0, pt)
        enc_next = jnp.where(pt_new == 1.0, e_if1, e_if0)
        return (h_new, pt_new, enc_next, encoded)

    init = (h0, pt0, enc0, enc0)
    h, _pt, _enc_unused, enc = jax.lax.fori_loop(0, _STEPS, step, init)

    eh = jnp.concatenate([enc.astype(bf16), h.astype(bf16)], axis=1)
    s_part = _db(eh, p1w_ref[0:2 * _L, :].astype(bf16))
    d_part = _db(eh, p1w_ref[2 * _L:, :].astype(bf16))
    pe = jnp.maximum(
        s_part[:, None, :] + d_part[None, :, :] + pb1_ref[:, :][None, :, :],
        0.0)
    out_ref[:, :] = _db(pe.reshape(E, _L).astype(bf16),
                        p2w_ref[:, :].astype(bf16)) + pb2_ref[:, :]


def kernel(X, enc_W, enc_b, M1_W, M2_W, U_W, mst_W, mst_b,
           pred_W1, pred_b1, pred_W2, pred_b2):
    args = (
        X[:, 0].reshape(_N, 1, 1), X[:, 1].reshape(_N, 1, 1),
        X[:, 0].reshape(1, _N, 1), X[:, 1].reshape(1, _N, 1),
        enc_W, enc_b.reshape(1, _L),
        M1_W, M2_W, U_W,
        mst_W, mst_b.reshape(1, 1),
        pred_W1, pred_b1.reshape(1, _L),
        pred_W2, pred_b2.reshape(1, 1),
    )
    out = pl.pallas_call(
        _prims_kernel,
        out_shape=jax.ShapeDtypeStruct((_N * _N, 1), jnp.float32),
        scratch_shapes=[pltpu.VMEM((_N * _N, 2 * _L + 1), jnp.bfloat16)],
    )(*args)
    return out.reshape(_N, _N)
